# Initial kernel scaffold; baseline (speedup 1.0000x reference)
import jax
import jax.numpy as jnp
from jax.experimental import pallas as pl


def kernel(input, embedding):
    raise NotImplementedError("write your pallas kernel here")


# speedup vs baseline: 1.0749x; 1.0749x over previous
import functools

import jax
import jax.numpy as jnp
from jax import lax
from jax.experimental import pallas as pl
from jax.experimental.pallas import tpu as pltpu
from jax.experimental.pallas import tpu_sc as plsc

N = 8192
K = 8192
D = 256

R = 512
CHUNK = 2048
CHUNKS = [(c, min(CHUNK, K - c)) for c in range(0, K, CHUNK)]


def _argmin_body(flat_ref, embt_ref, fl2_ref, e2_ref, idx_ref):
    f = flat_ref[:].astype(jnp.bfloat16)
    fl2 = fl2_ref[:]

    macc = jnp.full((R, 1), jnp.inf, jnp.float32)
    iacc = jnp.zeros((R, 1), jnp.int32)
    for c0, cw in CHUNKS:
        e_blk = embt_ref[:, pl.ds(c0, cw)].astype(jnp.bfloat16)
        e2 = e2_ref[:, pl.ds(c0, cw)]
        dot = jax.lax.dot_general(
            f, e_blk, (((1,), (0,)), ((), ())),
            preferred_element_type=jnp.float32,
        )
        scores = fl2 + e2 - 2.0 * dot
        m = jnp.min(scores, axis=1, keepdims=True)
        col = lax.broadcasted_iota(jnp.int32, (R, cw), 1)
        a = jnp.min(jnp.where(scores == m, col + c0, K), axis=1,
                    keepdims=True)
        better = m < macc
        iacc = jnp.where(better, a, iacc)
        macc = jnp.where(better, m, macc).astype(jnp.bfloat16).astype(jnp.float32)
    idx_ref[:] = iacc


def _tc_argmin(flat, embt, fl2, e2):
    return pl.pallas_call(
        _argmin_body,
        grid=(N // R,),
        in_specs=[
            pl.BlockSpec((R, D), lambda i: (i, 0)),
            pl.BlockSpec((D, K), lambda i: (0, 0)),
            pl.BlockSpec((R, 1), lambda i: (i, 0)),
            pl.BlockSpec((1, K), lambda i: (0, 0)),
        ],
        out_specs=pl.BlockSpec((R, 1), lambda i: (i, 0)),
        out_shape=jax.ShapeDtypeStruct((N, 1), jnp.int32),
    )(flat, embt, fl2, e2)


def _make_sc_gather():
    info = plsc.get_sparse_core_info()
    nc, ns = info.num_cores, info.num_subcores
    nw = nc * ns
    b_per_w = N // nw
    mesh = plsc.VectorSubcoreMesh(core_axis_name="c", subcore_axis_name="s")

    @functools.partial(
        pl.kernel, mesh=mesh,
        out_type=jax.ShapeDtypeStruct((N, D), jnp.float32),
        scratch_types=[
            pltpu.VMEM((b_per_w,), jnp.int32),
            pltpu.VMEM((b_per_w, D), jnp.float32),
            pltpu.SemaphoreType.DMA,
        ],
    )
    def gather_k(table_hbm, idx_hbm, out_hbm, idx_v, rows_v, sem):
        wid = lax.axis_index("s") * nc + lax.axis_index("c")
        base = wid * b_per_w
        pltpu.sync_copy(idx_hbm.at[pl.ds(base, b_per_w)], idx_v)
        pltpu.async_copy(table_hbm.at[idx_v], rows_v, sem).wait()
        pltpu.sync_copy(rows_v, out_hbm.at[pl.ds(base, b_per_w)])

    return gather_k


_sc_gather = None


def kernel(input, embedding):
    global _sc_gather
    if _sc_gather is None:
        _sc_gather = _make_sc_gather()
    flat = input.reshape(-1, D)
    fl2 = jnp.sum(flat ** 2, axis=1, keepdims=True)
    e2 = jnp.sum(embedding ** 2, axis=1).reshape(1, K)
    idx = _tc_argmin(flat, embedding.T, fl2, e2).reshape(-1)
    quant = _sc_gather(embedding, idx)
    return quant.reshape(input.shape)
